# initial kernel scaffold (unmeasured)
import jax
import jax.numpy as jnp
from jax import lax
from jax.experimental import pallas as pl
from jax.experimental.pallas import tpu as pltpu

N_DEV = 8
SQ = 1024
DM = 1024
HQ = 8
DH = 128
DW = HQ * DH
CHUNK = SQ // N_DEV
SCALE = 0.08838834764831843
BLK = 64


def kernel(x, Wq, K_ext, V_ext, Wo):
    def body(
        x_ref, wq_hbm, k_ref, v_ref, wo_hbm, out_ref,
        wq_vmem, wo_vmem, ctx_ref, partial_ref, red_ref,
        rs_recv, ag_recv,
        copy_sems,
        rs_send_sems, rs_recv_sems, ag_send_sems, ag_recv_sems,
    ):
        pos = lax.axis_index("i")

        wq_copy = pltpu.make_async_copy(
            wq_hbm.at[:, pl.ds(pos * DW, DW)], wq_vmem, copy_sems.at[0]
        )
        wo_copy = pltpu.make_async_copy(
            wo_hbm.at[pl.ds(pos * DW, DW), :], wo_vmem, copy_sems.at[1]
        )
        wq_copy.start()
        wo_copy.start()

        barrier_sem = pltpu.get_barrier_semaphore()
        for d in range(1, N_DEV):
            pl.semaphore_signal(
                barrier_sem, inc=1,
                device_id=((pos + d) % N_DEV,),
                device_id_type=pl.DeviceIdType.MESH,
            )
        pl.semaphore_wait(barrier_sem, N_DEV - 1)

        wq_copy.wait()
        q = jnp.dot(x_ref[0], wq_vmem[...], preferred_element_type=jnp.float32)

        qb = lax.broadcasted_iota(jnp.int32, (SQ, SQ), 0) // BLK
        kb = lax.broadcasted_iota(jnp.int32, (SQ, SQ), 1) // BLK
        bias = jnp.where((qb % 4) == (kb % 4), 0.0, -1e9).astype(jnp.float32)

        for h in range(HQ):
            qh = q[:, h * DH:(h + 1) * DH]
            kh = k_ref[0, :, h, :]
            s = lax.dot_general(
                qh, kh, (((1,), (1,)), ((), ())),
                preferred_element_type=jnp.float32,
            ) * SCALE + bias
            m = jnp.max(s, axis=1, keepdims=True)
            w = jnp.exp(s - m)
            w = w / jnp.sum(w, axis=1, keepdims=True)
            ctx_ref[:, h * DH:(h + 1) * DH] = jnp.dot(
                w, v_ref[0, :, h, :], preferred_element_type=jnp.float32
            )

        wo_copy.wait()
        partial_ref[...] = jnp.dot(
            ctx_ref[...], wo_vmem[...], preferred_element_type=jnp.float32
        )

        rs_rdmas = []
        for d in range(1, N_DEV):
            tgt = (pos + d) % N_DEV
            rdma = pltpu.make_async_remote_copy(
                src_ref=partial_ref.at[pl.ds(tgt * CHUNK, CHUNK)],
                dst_ref=rs_recv.at[d - 1],
                send_sem=rs_send_sems.at[d - 1],
                recv_sem=rs_recv_sems.at[d - 1],
                device_id=(tgt,),
                device_id_type=pl.DeviceIdType.MESH,
            )
            rdma.start()
            rs_rdmas.append(rdma)

        red = partial_ref[pl.ds(pos * CHUNK, CHUNK), :]
        for d in range(1, N_DEV):
            rs_rdmas[d - 1].wait_recv()
            red = red + rs_recv[d - 1]
        red_ref[...] = red

        ag_rdmas = []
        for d in range(1, N_DEV):
            tgt = (pos + d) % N_DEV
            rdma = pltpu.make_async_remote_copy(
                src_ref=red_ref,
                dst_ref=ag_recv.at[d - 1],
                send_sem=ag_send_sems.at[d - 1],
                recv_sem=ag_recv_sems.at[d - 1],
                device_id=(tgt,),
                device_id_type=pl.DeviceIdType.MESH,
            )
            rdma.start()
            ag_rdmas.append(rdma)

        out_ref[0, pl.ds(pos * CHUNK, CHUNK), :] = red_ref[...]
        for d in range(1, N_DEV):
            src = (pos - d) % N_DEV
            ag_rdmas[d - 1].wait_recv()
            out_ref[0, pl.ds(src * CHUNK, CHUNK), :] = ag_recv[d - 1]

        for d in range(1, N_DEV):
            rs_rdmas[d - 1].wait_send()
            ag_rdmas[d - 1].wait_send()

    return pl.pallas_call(
        body,
        out_shape=jax.ShapeDtypeStruct((1, SQ, DM), jnp.float32),
        in_specs=[
            pl.BlockSpec(memory_space=pltpu.VMEM),
            pl.BlockSpec(memory_space=pltpu.ANY),
            pl.BlockSpec(memory_space=pltpu.VMEM),
            pl.BlockSpec(memory_space=pltpu.VMEM),
            pl.BlockSpec(memory_space=pltpu.ANY),
        ],
        out_specs=pl.BlockSpec(memory_space=pltpu.VMEM),
        scratch_shapes=[
            pltpu.VMEM((SQ, DW), jnp.float32),
            pltpu.VMEM((DW, DM), jnp.float32),
            pltpu.VMEM((SQ, DW), jnp.float32),
            pltpu.VMEM((SQ, DM), jnp.float32),
            pltpu.VMEM((CHUNK, DM), jnp.float32),
            pltpu.VMEM((N_DEV - 1, CHUNK, DM), jnp.float32),
            pltpu.VMEM((N_DEV - 1, CHUNK, DM), jnp.float32),
            pltpu.SemaphoreType.DMA((2,)),
            pltpu.SemaphoreType.DMA((N_DEV - 1,)),
            pltpu.SemaphoreType.DMA((N_DEV - 1,)),
            pltpu.SemaphoreType.DMA((N_DEV - 1,)),
            pltpu.SemaphoreType.DMA((N_DEV - 1,)),
        ],
        compiler_params=pltpu.CompilerParams(collective_id=0),
    )(x, Wq, K_ext, V_ext, Wo)


# baseline (device time: 100490 ns/iter reference)
import jax
import jax.numpy as jnp
from jax import lax
from jax.experimental import pallas as pl
from jax.experimental.pallas import tpu as pltpu

N_DEV = 8
SQ = 1024
DM = 1024
HQ = 8
DH = 128
DW = HQ * DH
CHUNK = SQ // N_DEV
SCALE = 0.08838834764831843
BLK = 64


def kernel(x, Wq, K_ext, V_ext, Wo):
    def body(
        x_ref, wq_hbm, k_ref, v_ref, wo_hbm, out_ref,
        wq_vmem, wo_vmem, ctx_ref, partial_ref, red_ref,
        rs_recv, ag_recv,
        copy_sems,
        rs_send_sems, rs_recv_sems, ag_send_sems, ag_recv_sems,
    ):
        pos = lax.axis_index("i")

        wq_copy = pltpu.make_async_copy(
            wq_hbm.at[:, pl.ds(pos * DW, DW)], wq_vmem, copy_sems.at[0]
        )
        wo_copy = pltpu.make_async_copy(
            wo_hbm.at[pl.ds(pos * DW, DW), :], wo_vmem, copy_sems.at[1]
        )
        wq_copy.start()
        wo_copy.start()

        barrier_sem = pltpu.get_barrier_semaphore()
        for d in range(1, N_DEV):
            pl.semaphore_signal(
                barrier_sem, inc=1,
                device_id=((pos + d) % N_DEV,),
                device_id_type=pl.DeviceIdType.MESH,
            )
        pl.semaphore_wait(barrier_sem, N_DEV - 1)

        wq_copy.wait()
        q = jnp.dot(x_ref[0], wq_vmem[...], preferred_element_type=jnp.float32)

        qb = lax.broadcasted_iota(jnp.int32, (SQ, SQ), 0) // BLK
        kb = lax.broadcasted_iota(jnp.int32, (SQ, SQ), 1) // BLK
        bias = jnp.where((qb % 4) == (kb % 4), 0.0, -1e9).astype(jnp.float32)

        for h in range(HQ):
            qh = q[:, h * DH:(h + 1) * DH]
            kh = k_ref[0, :, h, :]
            s = lax.dot_general(
                qh, kh, (((1,), (1,)), ((), ())),
                preferred_element_type=jnp.float32,
            ) * SCALE + bias
            m = jnp.max(s, axis=1, keepdims=True)
            w = jnp.exp(s - m)
            w = w / jnp.sum(w, axis=1, keepdims=True)
            ctx_ref[:, h * DH:(h + 1) * DH] = jnp.dot(
                w, v_ref[0, :, h, :], preferred_element_type=jnp.float32
            )

        wo_copy.wait()
        partial_ref[...] = jnp.dot(
            ctx_ref[...], wo_vmem[...], preferred_element_type=jnp.float32
        )

        rs_rdmas = []
        for d in range(1, N_DEV):
            tgt = (pos + d) % N_DEV
            rdma = pltpu.make_async_remote_copy(
                src_ref=partial_ref.at[pl.ds(tgt * CHUNK, CHUNK)],
                dst_ref=rs_recv.at[d - 1],
                send_sem=rs_send_sems.at[d - 1],
                recv_sem=rs_recv_sems.at[d - 1],
                device_id=(tgt,),
                device_id_type=pl.DeviceIdType.MESH,
            )
            rdma.start()
            rs_rdmas.append(rdma)

        red = partial_ref[pl.ds(pos * CHUNK, CHUNK), :]
        for d in range(1, N_DEV):
            rs_rdmas[d - 1].wait_recv()
            red = red + rs_recv[d - 1]
        red_ref[...] = red

        ag_rdmas = []
        for d in range(1, N_DEV):
            tgt = (pos + d) % N_DEV
            rdma = pltpu.make_async_remote_copy(
                src_ref=red_ref,
                dst_ref=ag_recv.at[d - 1],
                send_sem=ag_send_sems.at[d - 1],
                recv_sem=ag_recv_sems.at[d - 1],
                device_id=(tgt,),
                device_id_type=pl.DeviceIdType.MESH,
            )
            rdma.start()
            ag_rdmas.append(rdma)

        out_ref[0, pl.ds(pos * CHUNK, CHUNK), :] = red_ref[...]
        for d in range(1, N_DEV):
            src = (pos - d) % N_DEV
            ag_rdmas[d - 1].wait_recv()
            out_ref[0, pl.ds(src * CHUNK, CHUNK), :] = ag_recv[d - 1]

        for d in range(1, N_DEV):
            rs_rdmas[d - 1].wait_send()
            ag_rdmas[d - 1].wait_send()

    return pl.pallas_call(
        body,
        out_shape=jax.ShapeDtypeStruct((1, SQ, DM), jnp.float32),
        in_specs=[
            pl.BlockSpec(memory_space=pltpu.MemorySpace.VMEM),
            pl.BlockSpec(memory_space=pltpu.MemorySpace.HBM),
            pl.BlockSpec(memory_space=pltpu.MemorySpace.VMEM),
            pl.BlockSpec(memory_space=pltpu.MemorySpace.VMEM),
            pl.BlockSpec(memory_space=pltpu.MemorySpace.HBM),
        ],
        out_specs=pl.BlockSpec(memory_space=pltpu.MemorySpace.VMEM),
        scratch_shapes=[
            pltpu.VMEM((SQ, DW), jnp.float32),
            pltpu.VMEM((DW, DM), jnp.float32),
            pltpu.VMEM((SQ, DW), jnp.float32),
            pltpu.VMEM((SQ, DM), jnp.float32),
            pltpu.VMEM((CHUNK, DM), jnp.float32),
            pltpu.VMEM((N_DEV - 1, CHUNK, DM), jnp.float32),
            pltpu.VMEM((N_DEV - 1, CHUNK, DM), jnp.float32),
            pltpu.SemaphoreType.DMA((2,)),
            pltpu.SemaphoreType.DMA((N_DEV - 1,)),
            pltpu.SemaphoreType.DMA((N_DEV - 1,)),
            pltpu.SemaphoreType.DMA((N_DEV - 1,)),
            pltpu.SemaphoreType.DMA((N_DEV - 1,)),
        ],
        compiler_params=pltpu.CompilerParams(collective_id=0),
    )(x, Wq, K_ext, V_ext, Wo)


# device time: 80473 ns/iter; 1.2487x vs baseline; 1.2487x over previous
import jax
import jax.numpy as jnp
from jax import lax
from jax.experimental import pallas as pl
from jax.experimental.pallas import tpu as pltpu

N_DEV = 8
SQ = 1024
DM = 1024
HQ = 8
DH = 128
DW = HQ * DH
CHUNK = SQ // N_DEV
HALF = SQ // 2
SCALE = 0.08838834764831843
BLK = 64


def kernel(x, Wq, K_ext, V_ext, Wo):
    def body(
        x_ref, wq_hbm, k_ref, v_ref, wo_hbm, out_ref,
        wq_vmem, wo_vmem, partial_ref, red_ref,
        rs_recv, ag_recv,
        copy_sems,
        rs_send_sems, rs_recv_sems, ag_send_sems, ag_recv_sems,
    ):
        pos = lax.axis_index("i")

        wq_copy = pltpu.make_async_copy(
            wq_hbm.at[:, pl.ds(pos * DW, DW)], wq_vmem, copy_sems.at[0]
        )
        wo_copy = pltpu.make_async_copy(
            wo_hbm.at[pl.ds(pos * DW, DW), :], wo_vmem, copy_sems.at[1]
        )
        wq_copy.start()
        wo_copy.start()

        barrier_sem = pltpu.get_barrier_semaphore()
        for d in range(1, N_DEV):
            pl.semaphore_signal(
                barrier_sem, inc=1,
                device_id=((pos + d) % N_DEV,),
                device_id_type=pl.DeviceIdType.MESH,
            )
        pl.semaphore_wait(barrier_sem, N_DEV - 1)

        wq_copy.wait()
        wo_copy.wait()

        kbm = (lax.broadcasted_iota(jnp.int32, (HALF, SQ), 1) // BLK) % 4

        rs_rdmas = []
        for half in range(2):
            r0 = half * HALF
            xh = x_ref[0, r0:r0 + HALF, :]
            qh = jnp.dot(xh, wq_vmem[...], preferred_element_type=jnp.float32)
            qbm = ((lax.broadcasted_iota(jnp.int32, (HALF, SQ), 0) + r0)
                   // BLK) % 4
            bias = jnp.where(qbm == kbm, 0.0, -1e9).astype(jnp.float32)
            ctx = []
            for h in range(HQ):
                s = lax.dot_general(
                    qh[:, h * DH:(h + 1) * DH], k_ref[0, :, h, :],
                    (((1,), (1,)), ((), ())),
                    preferred_element_type=jnp.float32,
                ) * SCALE + bias
                m = jnp.max(s, axis=1, keepdims=True)
                w = jnp.exp(s - m)
                w = w / jnp.sum(w, axis=1, keepdims=True)
                ctx.append(jnp.dot(
                    w, v_ref[0, :, h, :], preferred_element_type=jnp.float32
                ))
            ph = jnp.dot(
                jnp.concatenate(ctx, axis=1), wo_vmem[...],
                preferred_element_type=jnp.float32,
            )
            partial_ref[r0:r0 + HALF, :] = ph.astype(jnp.bfloat16)

            for t in range(half * 4, half * 4 + 4):
                dd = (t - pos) % N_DEV
                rdma = pltpu.make_async_remote_copy(
                    src_ref=partial_ref.at[pl.ds(t * CHUNK, CHUNK)],
                    dst_ref=rs_recv.at[dd],
                    send_sem=rs_send_sems.at[dd],
                    recv_sem=rs_recv_sems.at[dd],
                    device_id=(t,),
                    device_id_type=pl.DeviceIdType.MESH,
                )
                rdma.start()
                rs_rdmas.append(rdma)

        for r in rs_rdmas:
            r.wait_recv()
        red = rs_recv[0].astype(jnp.float32)
        for d in range(1, N_DEV):
            red = red + rs_recv[d].astype(jnp.float32)
        red_ref[...] = red.astype(jnp.bfloat16)

        ag_rdmas = []
        for d in range(1, N_DEV):
            tgt = (pos + d) % N_DEV
            rdma = pltpu.make_async_remote_copy(
                src_ref=red_ref,
                dst_ref=ag_recv.at[d - 1],
                send_sem=ag_send_sems.at[d - 1],
                recv_sem=ag_recv_sems.at[d - 1],
                device_id=(tgt,),
                device_id_type=pl.DeviceIdType.MESH,
            )
            rdma.start()
            ag_rdmas.append(rdma)

        out_ref[0, pl.ds(pos * CHUNK, CHUNK), :] = red
        for d in range(1, N_DEV):
            src = (pos - d) % N_DEV
            ag_rdmas[d - 1].wait_recv()
            out_ref[0, pl.ds(src * CHUNK, CHUNK), :] = (
                ag_recv[d - 1].astype(jnp.float32)
            )

        for r in rs_rdmas:
            r.wait_send()
        for r in ag_rdmas:
            r.wait_send()

    return pl.pallas_call(
        body,
        out_shape=jax.ShapeDtypeStruct((1, SQ, DM), jnp.float32),
        in_specs=[
            pl.BlockSpec(memory_space=pltpu.MemorySpace.VMEM),
            pl.BlockSpec(memory_space=pltpu.MemorySpace.HBM),
            pl.BlockSpec(memory_space=pltpu.MemorySpace.VMEM),
            pl.BlockSpec(memory_space=pltpu.MemorySpace.VMEM),
            pl.BlockSpec(memory_space=pltpu.MemorySpace.HBM),
        ],
        out_specs=pl.BlockSpec(memory_space=pltpu.MemorySpace.VMEM),
        scratch_shapes=[
            pltpu.VMEM((SQ, DW), jnp.float32),
            pltpu.VMEM((DW, DM), jnp.float32),
            pltpu.VMEM((SQ, DM), jnp.bfloat16),
            pltpu.VMEM((CHUNK, DM), jnp.bfloat16),
            pltpu.VMEM((N_DEV, CHUNK, DM), jnp.bfloat16),
            pltpu.VMEM((N_DEV - 1, CHUNK, DM), jnp.bfloat16),
            pltpu.SemaphoreType.DMA((2,)),
            pltpu.SemaphoreType.DMA((N_DEV,)),
            pltpu.SemaphoreType.DMA((N_DEV,)),
            pltpu.SemaphoreType.DMA((N_DEV - 1,)),
            pltpu.SemaphoreType.DMA((N_DEV - 1,)),
        ],
        compiler_params=pltpu.CompilerParams(collective_id=0),
    )(x, Wq, K_ext, V_ext, Wo)
